# Initial kernel scaffold; baseline (speedup 1.0000x reference)
import jax
import jax.numpy as jnp
from jax.experimental import pallas as pl


def kernel(x, edge_index, batch, bn0_g, bn0_b, W1, b1, bn1_g, bn1_b, W2, b2, bn2_g, bn2_b, W3, b3, bn3_g, bn3_b, L1_W, L1_b, L2_W, L2_b):
    raise NotImplementedError("write your pallas kernel here")


# speedup vs baseline: 13.8982x; 13.8982x over previous
import functools

import jax
import jax.numpy as jnp
from jax import lax
from jax.experimental import pallas as pl
from jax.experimental.pallas import tpu as pltpu
from jax.experimental.pallas import tpu_sc as plsc

EPS = 1e-5
G = 256
NC = 2
NS = 16
NW = NC * NS
LANE = 128
KC = 32

f32 = jnp.float32
i32 = jnp.int32

_SC_PARAMS = pltpu.CompilerParams(use_tc_tiling_on_sc=False)


def _mesh2():
    return plsc.VectorSubcoreMesh(core_axis_name="c", subcore_axis_name="s")


def _make_deg_kernel(npad, rpw, slice_rows):
    @functools.partial(
        pl.kernel,
        out_type=jax.ShapeDtypeStruct((NC, npad, 8), f32),
        mesh=_mesh2(),
        name="deg_hist",
        compiler_params=_SC_PARAMS,
        scratch_types=[
            pltpu.VMEM((rpw, LANE), i32),
            pltpu.VMEM((LANE, 8), f32),
            pltpu.VMEM_SHARED((npad, 8), f32),
        ],
    )
    def deg_kernel(dst_hbm, ones_hbm, zin_hbm, out_hbm, dst_v, ones_v, z_sh):
        c = lax.axis_index("c")
        s = lax.axis_index("s")
        wid = c * NS + s
        pltpu.sync_copy(zin_hbm, z_sh.at[pl.ds(s * slice_rows, slice_rows), :])
        pltpu.sync_copy(ones_hbm, ones_v)
        plsc.subcore_barrier()
        pltpu.sync_copy(dst_hbm.at[wid], dst_v)

        def body(j, carry):
            pltpu.sync_copy(ones_v, z_sh.at[dst_v.at[j]], add=True)
            return carry

        lax.fori_loop(0, rpw, body, 0)
        plsc.subcore_barrier()
        pltpu.sync_copy(
            z_sh.at[pl.ds(s * slice_rows, slice_rows), :],
            out_hbm.at[c, pl.ds(s * slice_rows, slice_rows), :],
        )

    return deg_kernel


def _make_prop_kernel(npad, w, rpw, slice_rows, name):
    nchunks = rpw // KC

    @functools.partial(
        pl.kernel,
        out_type=jax.ShapeDtypeStruct((NC, npad, w), f32),
        mesh=_mesh2(),
        name=name,
        compiler_params=_SC_PARAMS,
        scratch_types=[
            pltpu.VMEM((KC, LANE), i32),
            pltpu.VMEM((KC, LANE), i32),
            pltpu.VMEM((LANE, w), f32),
            pltpu.VMEM_SHARED((npad, w), f32),
            pltpu.SemaphoreType.DMA,
        ],
    )
    def prop_kernel(y_hbm, src_hbm, dst_hbm, zin_hbm, out_hbm,
                    src_v, dst_v, rows_v, z_sh, sem):
        c = lax.axis_index("c")
        s = lax.axis_index("s")
        wid = c * NS + s
        pltpu.sync_copy(zin_hbm, z_sh.at[pl.ds(s * slice_rows, slice_rows), :])
        plsc.subcore_barrier()

        def chunk(k, carry):
            pltpu.sync_copy(src_hbm.at[wid, pl.ds(k * KC, KC)], src_v)
            pltpu.sync_copy(dst_hbm.at[wid, pl.ds(k * KC, KC)], dst_v)

            def body(j, carry2):
                pltpu.async_copy(y_hbm.at[src_v.at[j]], rows_v, sem).wait()
                pltpu.sync_copy(rows_v, z_sh.at[dst_v.at[j]], add=True)
                return carry2

            lax.fori_loop(0, KC, body, 0)
            return carry

        lax.fori_loop(0, nchunks, chunk, 0)
        plsc.subcore_barrier()
        pltpu.sync_copy(
            z_sh.at[pl.ds(s * slice_rows, slice_rows), :],
            out_hbm.at[c, pl.ds(s * slice_rows, slice_rows), :],
        )

    return prop_kernel


def _make_pool_kernel(rows_per_worker):
    rw = rows_per_worker

    @functools.partial(
        pl.kernel,
        out_type=jax.ShapeDtypeStruct((NW, G, 32), f32),
        mesh=_mesh2(),
        name="pool_max",
        compiler_params=_SC_PARAMS,
        scratch_types=[
            pltpu.VMEM((rw + 16,), i32),
            pltpu.VMEM((rw, 32), f32),
            pltpu.VMEM((G, 32), f32),
        ],
    )
    def pool_kernel(h_hbm, batch_hbm, out_hbm, ids_v, h_v, acc_v):
        c = lax.axis_index("c")
        s = lax.axis_index("s")
        wid = c * NS + s
        pltpu.sync_copy(batch_hbm.at[wid, 0], ids_v.at[pl.ds(0, rw)])
        pltpu.sync_copy(h_hbm.at[wid], h_v)

        neg = jnp.full((16,), -jnp.inf, f32)

        def init(r, carry):
            acc_v[r, pl.ds(0, 16)] = neg
            acc_v[r, pl.ds(16, 16)] = neg
            return carry

        lax.fori_loop(0, G, init, 0)

        def body(r, carry):
            g = ids_v[pl.ds(r, 16)][0]
            a0 = acc_v[g, pl.ds(0, 16)]
            a1 = acc_v[g, pl.ds(16, 16)]
            h0 = h_v[r, pl.ds(0, 16)]
            h1 = h_v[r, pl.ds(16, 16)]
            acc_v[g, pl.ds(0, 16)] = jnp.maximum(a0, h0)
            acc_v[g, pl.ds(16, 16)] = jnp.maximum(a1, h1)
            return carry

        lax.fori_loop(0, rw, body, 0)
        pltpu.sync_copy(acc_v, out_hbm.at[wid])

    return pool_kernel


def _vpu_mm(a, w_ref, k):
    acc = a[:, 0:1] * w_ref[0, :][None, :]
    for j in range(1, k):
        acc = acc + a[:, j:j + 1] * w_ref[j, :][None, :]
    return acc


def _stats_kernel(n, w, blk):
    nb = n // blk

    def body(x_ref, s_ref):
        i = pl.program_id(0)

        @pl.when(i == 0)
        def _():
            s_ref[...] = jnp.zeros_like(s_ref)

        xb = x_ref[...]
        ssum = jnp.sum(xb, axis=0, keepdims=True)
        ssq = jnp.sum(xb * xb, axis=0, keepdims=True)
        s_ref[...] += jnp.concatenate([ssum, ssq], axis=0)

    return pl.pallas_call(
        body,
        grid=(nb,),
        in_specs=[pl.BlockSpec((blk, w), lambda i: (i, 0))],
        out_specs=pl.BlockSpec((2, w), lambda i: (0, 0)),
        out_shape=jax.ShapeDtypeStruct((2, w), f32),
    )


def _b0_kernel(n, blk):
    nb = n // blk

    def body(x_ref, xs_ref, deg_ref, g_ref, b_ref, y_ref, dinv_ref):
        nn = f32(n)
        m = xs_ref[0, 0] / nn
        v = xs_ref[1, 0] / nn - m * m
        d = deg_ref[0][:, 0:1] + deg_ref[1][:, 0:1] + 1.0
        dinv = lax.rsqrt(d)
        h = g_ref[0, 0] * (x_ref[...] - m) * lax.rsqrt(v + EPS) + b_ref[0, 0]
        dinv_ref[...] = dinv
        y = dinv * h
        y_ref[...] = jnp.concatenate(
            [y, jnp.zeros((y.shape[0], 7), f32)], axis=1)

    return pl.pallas_call(
        body,
        grid=(nb,),
        in_specs=[
            pl.BlockSpec((blk, 1), lambda i: (i, 0)),
            pl.BlockSpec((2, 1), lambda i: (0, 0)),
            pl.BlockSpec((2, blk, 8), lambda i: (0, i, 0)),
            pl.BlockSpec((1, 1), lambda i: (0, 0)),
            pl.BlockSpec((1, 1), lambda i: (0, 0)),
        ],
        out_specs=[
            pl.BlockSpec((blk, 8), lambda i: (i, 0)),
            pl.BlockSpec((blk, 1), lambda i: (i, 0)),
        ],
        out_shape=[
            jax.ShapeDtypeStruct((n, 8), f32),
            jax.ShapeDtypeStruct((n, 1), f32),
        ],
    )


def _a_kernel(n, w_in, w_use, w_out, blk):
    nb = n // blk

    def body(z_ref, y_ref, dinv_ref, w_ref, b_ref, g_ref, s_ref):
        i = pl.program_id(0)
        a = dinv_ref[...] * (z_ref[0] + z_ref[1] + y_ref[...])
        g = _vpu_mm(a, w_ref, w_use) + b_ref[...]
        g_ref[...] = g

        @pl.when(i == 0)
        def _():
            s_ref[...] = jnp.zeros_like(s_ref)

        ssum = jnp.sum(g, axis=0, keepdims=True)
        ssq = jnp.sum(g * g, axis=0, keepdims=True)
        s_ref[...] += jnp.concatenate([ssum, ssq], axis=0)

    return pl.pallas_call(
        body,
        grid=(nb,),
        in_specs=[
            pl.BlockSpec((2, blk, w_in), lambda i: (0, i, 0)),
            pl.BlockSpec((blk, w_in), lambda i: (i, 0)),
            pl.BlockSpec((blk, 1), lambda i: (i, 0)),
            pl.BlockSpec((w_use, w_out), lambda i: (0, 0)),
            pl.BlockSpec((1, w_out), lambda i: (0, 0)),
        ],
        out_specs=[
            pl.BlockSpec((blk, w_out), lambda i: (i, 0)),
            pl.BlockSpec((2, w_out), lambda i: (0, 0)),
        ],
        out_shape=[
            jax.ShapeDtypeStruct((n, w_out), f32),
            jax.ShapeDtypeStruct((2, w_out), f32),
        ],
    )


def _b_kernel(n, w, blk, relu, dinv_scale):
    nb = n // blk

    def body(g_ref, s_ref, gam_ref, bet_ref, dinv_ref, y_ref):
        nn = f32(n)
        m = s_ref[0:1, :] / nn
        v = s_ref[1:2, :] / nn - m * m
        h = gam_ref[...] * (g_ref[...] - m) * lax.rsqrt(v + EPS) + bet_ref[...]
        if relu:
            h = jnp.maximum(h, 0.0)
        if dinv_scale:
            h = dinv_ref[...] * h
        y_ref[...] = h

    return pl.pallas_call(
        body,
        grid=(nb,),
        in_specs=[
            pl.BlockSpec((blk, w), lambda i: (i, 0)),
            pl.BlockSpec((2, w), lambda i: (0, 0)),
            pl.BlockSpec((1, w), lambda i: (0, 0)),
            pl.BlockSpec((1, w), lambda i: (0, 0)),
            pl.BlockSpec((blk, 1), lambda i: (i, 0)),
        ],
        out_specs=pl.BlockSpec((blk, w), lambda i: (i, 0)),
        out_shape=jax.ShapeDtypeStruct((n, w), f32),
    )


def _mlp_kernel():
    def body(p_ref, w1_ref, b1_ref, w2_ref, b2_ref, o_ref):
        p = jnp.max(p_ref[...], axis=0)
        p = jnp.where(jnp.isfinite(p), p, 0.0)
        h = jnp.maximum(_vpu_mm(p, w1_ref, 32) + b1_ref[...], 0.0)
        o = jnp.maximum(_vpu_mm(h, w2_ref, 64) + b2_ref[...], 0.0)
        o_ref[...] = o

    return pl.pallas_call(
        body,
        out_shape=jax.ShapeDtypeStruct((G, 2), f32),
    )


def kernel(x, edge_index, batch, bn0_g, bn0_b, W1, b1, bn1_g, bn1_b,
           W2, b2, bn2_g, bn2_b, W3, b3, bn3_g, bn3_b, L1_W, L1_b, L2_W, L2_b):
    n = x.shape[0]
    e = edge_index.shape[1]

    npad = ((n + 1 + NS * 8 - 1) // (NS * 8)) * (NS * 8)
    slice_rows = npad // NS
    rows = -(-e // LANE)
    rows_pad = -(-rows // (NW * KC)) * (NW * KC)
    rpw = rows_pad // NW
    e_pad = rows_pad * LANE
    blk = 1000
    assert n % blk == 0 and n % NW == 0

    pad = e_pad - e
    src_f = jnp.concatenate([edge_index[0], jnp.zeros((pad,), i32)])
    dst_f = jnp.concatenate([edge_index[1], jnp.full((pad,), n, i32)])
    src_p = src_f.reshape(NW, rpw, LANE)
    dst_p = dst_f.reshape(NW, rpw, LANE)

    zin8 = jnp.zeros((slice_rows, 8), f32)
    zin16 = jnp.zeros((slice_rows, 16), f32)
    ones_lane = jnp.ones((LANE, 8), f32)

    deg2 = _make_deg_kernel(npad, rpw, slice_rows)(dst_p, ones_lane, zin8)
    xs = _stats_kernel(n, 1, blk)(x)
    y1, dinv = _b0_kernel(n, blk)(
        x, xs, deg2, bn0_g.reshape(1, 1), bn0_b.reshape(1, 1))

    z1 = _make_prop_kernel(npad, 8, rpw, slice_rows, "prop_l1")(
        y1, src_p, dst_p, zin8)
    g1, s1 = _a_kernel(n, 8, 1, 8, blk)(
        z1[:, :n], y1, dinv, W1, b1.reshape(1, 8))
    y2 = _b_kernel(n, 8, blk, True, True)(
        g1, s1, bn1_g.reshape(1, 8), bn1_b.reshape(1, 8), dinv)

    z2 = _make_prop_kernel(npad, 8, rpw, slice_rows, "prop_l2")(
        y2, src_p, dst_p, zin8)
    g2, s2 = _a_kernel(n, 8, 8, 16, blk)(
        z2[:, :n], y2, dinv, W2, b2.reshape(1, 16))
    y3 = _b_kernel(n, 16, blk, True, True)(
        g2, s2, bn2_g.reshape(1, 16), bn2_b.reshape(1, 16), dinv)

    z3 = _make_prop_kernel(npad, 16, rpw, slice_rows, "prop_l3")(
        y3, src_p, dst_p, zin16)
    g3, s3 = _a_kernel(n, 16, 16, 32, blk)(
        z3[:, :n], y3, dinv, W3, b3.reshape(1, 32))
    h3 = _b_kernel(n, 32, blk, True, False)(
        g3, s3, bn3_g.reshape(1, 32), bn3_b.reshape(1, 32), dinv)

    batch3 = batch.reshape(NW, 1, n // NW)
    h3r = h3.reshape(NW, n // NW, 32)
    pool_part = _make_pool_kernel(n // NW)(h3r, batch3)
    out = _mlp_kernel()(
        pool_part, L1_W, L1_b.reshape(1, 64), L2_W, L2_b.reshape(1, 2))
    return out


# speedup vs baseline: 14.1019x; 1.0147x over previous
import functools

import jax
import jax.numpy as jnp
from jax import lax
from jax.experimental import pallas as pl
from jax.experimental.pallas import tpu as pltpu
from jax.experimental.pallas import tpu_sc as plsc

EPS = 1e-5
G = 256
NC = 2
NS = 16
NW = NC * NS
LANE = 128
KC = 32

f32 = jnp.float32
i32 = jnp.int32

_SC_PARAMS = pltpu.CompilerParams(use_tc_tiling_on_sc=False)


def _mesh2():
    return plsc.VectorSubcoreMesh(core_axis_name="c", subcore_axis_name="s")


def _make_deg_kernel(npad, rpw, slice_rows):
    @functools.partial(
        pl.kernel,
        out_type=jax.ShapeDtypeStruct((NC, npad, 8), f32),
        mesh=_mesh2(),
        name="deg_hist",
        compiler_params=_SC_PARAMS,
        scratch_types=[
            pltpu.VMEM((rpw, LANE), i32),
            pltpu.VMEM((LANE, 8), f32),
            pltpu.VMEM_SHARED((npad, 8), f32),
            pltpu.SemaphoreType.DMA,
        ],
    )
    def deg_kernel(dst_hbm, ones_hbm, zin_hbm, out_hbm, dst_v, ones_v, z_sh,
                   sem_s):
        c = lax.axis_index("c")
        s = lax.axis_index("s")
        wid = c * NS + s
        pltpu.sync_copy(zin_hbm, z_sh.at[pl.ds(s * slice_rows, slice_rows), :])
        pltpu.sync_copy(ones_hbm, ones_v)
        plsc.subcore_barrier()
        pltpu.sync_copy(dst_hbm.at[wid], dst_v)

        def body(j, carry):
            sd = [pltpu.async_copy(ones_v, z_sh.at[dst_v.at[j * 8 + u]],
                                   sem_s, add=True) for u in range(8)]
            for d in sd:
                d.wait()
            return carry

        lax.fori_loop(0, rpw // 8, body, 0)
        plsc.subcore_barrier()
        pltpu.sync_copy(
            z_sh.at[pl.ds(s * slice_rows, slice_rows), :],
            out_hbm.at[c, pl.ds(s * slice_rows, slice_rows), :],
        )

    return deg_kernel


def _make_prop_kernel(npad, w, rpw, slice_rows, name):
    kcp = 8
    nchunks = rpw // kcp

    @functools.partial(
        pl.kernel,
        out_type=jax.ShapeDtypeStruct((NC, npad, w), f32),
        mesh=_mesh2(),
        name=name,
        compiler_params=_SC_PARAMS,
        scratch_types=[
            pltpu.VMEM((kcp, LANE), i32),
            pltpu.VMEM((kcp, LANE), i32),
            pltpu.VMEM((kcp, LANE, w), f32),
            pltpu.VMEM_SHARED((npad, w), f32),
            pltpu.SemaphoreType.DMA,
            pltpu.SemaphoreType.DMA,
        ],
    )
    def prop_kernel(y_hbm, src_hbm, dst_hbm, zin_hbm, out_hbm,
                    src_v, dst_v, rows_v, z_sh, sem_g, sem_s):
        c = lax.axis_index("c")
        s = lax.axis_index("s")
        wid = c * NS + s
        pltpu.sync_copy(zin_hbm, z_sh.at[pl.ds(s * slice_rows, slice_rows), :])
        plsc.subcore_barrier()

        def chunk(k, carry):
            pltpu.sync_copy(src_hbm.at[wid, pl.ds(k * kcp, kcp)], src_v)
            pltpu.sync_copy(dst_hbm.at[wid, pl.ds(k * kcp, kcp)], dst_v)
            gd = [pltpu.async_copy(y_hbm.at[src_v.at[j]], rows_v.at[j], sem_g)
                  for j in range(kcp)]
            sd = []
            for j in range(kcp):
                gd[j].wait()
                sd.append(pltpu.async_copy(
                    rows_v.at[j], z_sh.at[dst_v.at[j]], sem_s, add=True))
            for d in sd:
                d.wait()
            return carry

        lax.fori_loop(0, nchunks, chunk, 0)
        plsc.subcore_barrier()
        pltpu.sync_copy(
            z_sh.at[pl.ds(s * slice_rows, slice_rows), :],
            out_hbm.at[c, pl.ds(s * slice_rows, slice_rows), :],
        )

    return prop_kernel


def _make_pool_kernel(rows_per_worker):
    rw = rows_per_worker

    @functools.partial(
        pl.kernel,
        out_type=jax.ShapeDtypeStruct((NW, G, 32), f32),
        mesh=_mesh2(),
        name="pool_max",
        compiler_params=_SC_PARAMS,
        scratch_types=[
            pltpu.VMEM((rw + 16,), i32),
            pltpu.VMEM((rw, 32), f32),
            pltpu.VMEM((G, 32), f32),
        ],
    )
    def pool_kernel(h_hbm, batch_hbm, out_hbm, ids_v, h_v, acc_v):
        c = lax.axis_index("c")
        s = lax.axis_index("s")
        wid = c * NS + s
        pltpu.sync_copy(batch_hbm.at[wid, 0], ids_v.at[pl.ds(0, rw)])
        pltpu.sync_copy(h_hbm.at[wid], h_v)

        neg = jnp.full((16,), -jnp.inf, f32)

        def init(r, carry):
            acc_v[r, pl.ds(0, 16)] = neg
            acc_v[r, pl.ds(16, 16)] = neg
            return carry

        lax.fori_loop(0, G, init, 0)

        def body(r, carry):
            g = ids_v[pl.ds(r, 16)][0]
            a0 = acc_v[g, pl.ds(0, 16)]
            a1 = acc_v[g, pl.ds(16, 16)]
            h0 = h_v[r, pl.ds(0, 16)]
            h1 = h_v[r, pl.ds(16, 16)]
            acc_v[g, pl.ds(0, 16)] = jnp.maximum(a0, h0)
            acc_v[g, pl.ds(16, 16)] = jnp.maximum(a1, h1)
            return carry

        lax.fori_loop(0, rw, body, 0)
        pltpu.sync_copy(acc_v, out_hbm.at[wid])

    return pool_kernel


def _vpu_mm(a, w_ref, k):
    acc = a[:, 0:1] * w_ref[0, :][None, :]
    for j in range(1, k):
        acc = acc + a[:, j:j + 1] * w_ref[j, :][None, :]
    return acc


def _stats_kernel(n, w, blk):
    nb = n // blk

    def body(x_ref, s_ref):
        i = pl.program_id(0)

        @pl.when(i == 0)
        def _():
            s_ref[...] = jnp.zeros_like(s_ref)

        xb = x_ref[...]
        ssum = jnp.sum(xb, axis=0, keepdims=True)
        ssq = jnp.sum(xb * xb, axis=0, keepdims=True)
        s_ref[...] += jnp.concatenate([ssum, ssq], axis=0)

    return pl.pallas_call(
        body,
        grid=(nb,),
        in_specs=[pl.BlockSpec((blk, w), lambda i: (i, 0))],
        out_specs=pl.BlockSpec((2, w), lambda i: (0, 0)),
        out_shape=jax.ShapeDtypeStruct((2, w), f32),
    )


def _b0_kernel(n, blk):
    nb = n // blk

    def body(x_ref, xs_ref, deg_ref, g_ref, b_ref, y_ref, dinv_ref):
        nn = f32(n)
        m = xs_ref[0, 0] / nn
        v = xs_ref[1, 0] / nn - m * m
        d = deg_ref[0][:, 0:1] + deg_ref[1][:, 0:1] + 1.0
        dinv = lax.rsqrt(d)
        h = g_ref[0, 0] * (x_ref[...] - m) * lax.rsqrt(v + EPS) + b_ref[0, 0]
        dinv_ref[...] = dinv
        y = dinv * h
        y_ref[...] = jnp.concatenate(
            [y, jnp.zeros((y.shape[0], 7), f32)], axis=1)

    return pl.pallas_call(
        body,
        grid=(nb,),
        in_specs=[
            pl.BlockSpec((blk, 1), lambda i: (i, 0)),
            pl.BlockSpec((2, 1), lambda i: (0, 0)),
            pl.BlockSpec((2, blk, 8), lambda i: (0, i, 0)),
            pl.BlockSpec((1, 1), lambda i: (0, 0)),
            pl.BlockSpec((1, 1), lambda i: (0, 0)),
        ],
        out_specs=[
            pl.BlockSpec((blk, 8), lambda i: (i, 0)),
            pl.BlockSpec((blk, 1), lambda i: (i, 0)),
        ],
        out_shape=[
            jax.ShapeDtypeStruct((n, 8), f32),
            jax.ShapeDtypeStruct((n, 1), f32),
        ],
    )


def _a_kernel(n, w_in, w_use, w_out, blk):
    nb = n // blk

    def body(z_ref, y_ref, dinv_ref, w_ref, b_ref, g_ref, s_ref):
        i = pl.program_id(0)
        a = dinv_ref[...] * (z_ref[0] + z_ref[1] + y_ref[...])
        g = _vpu_mm(a, w_ref, w_use) + b_ref[...]
        g_ref[...] = g

        @pl.when(i == 0)
        def _():
            s_ref[...] = jnp.zeros_like(s_ref)

        ssum = jnp.sum(g, axis=0, keepdims=True)
        ssq = jnp.sum(g * g, axis=0, keepdims=True)
        s_ref[...] += jnp.concatenate([ssum, ssq], axis=0)

    return pl.pallas_call(
        body,
        grid=(nb,),
        in_specs=[
            pl.BlockSpec((2, blk, w_in), lambda i: (0, i, 0)),
            pl.BlockSpec((blk, w_in), lambda i: (i, 0)),
            pl.BlockSpec((blk, 1), lambda i: (i, 0)),
            pl.BlockSpec((w_use, w_out), lambda i: (0, 0)),
            pl.BlockSpec((1, w_out), lambda i: (0, 0)),
        ],
        out_specs=[
            pl.BlockSpec((blk, w_out), lambda i: (i, 0)),
            pl.BlockSpec((2, w_out), lambda i: (0, 0)),
        ],
        out_shape=[
            jax.ShapeDtypeStruct((n, w_out), f32),
            jax.ShapeDtypeStruct((2, w_out), f32),
        ],
    )


def _b_kernel(n, w, blk, relu, dinv_scale):
    nb = n // blk

    def body(g_ref, s_ref, gam_ref, bet_ref, dinv_ref, y_ref):
        nn = f32(n)
        m = s_ref[0:1, :] / nn
        v = s_ref[1:2, :] / nn - m * m
        h = gam_ref[...] * (g_ref[...] - m) * lax.rsqrt(v + EPS) + bet_ref[...]
        if relu:
            h = jnp.maximum(h, 0.0)
        if dinv_scale:
            h = dinv_ref[...] * h
        y_ref[...] = h

    return pl.pallas_call(
        body,
        grid=(nb,),
        in_specs=[
            pl.BlockSpec((blk, w), lambda i: (i, 0)),
            pl.BlockSpec((2, w), lambda i: (0, 0)),
            pl.BlockSpec((1, w), lambda i: (0, 0)),
            pl.BlockSpec((1, w), lambda i: (0, 0)),
            pl.BlockSpec((blk, 1), lambda i: (i, 0)),
        ],
        out_specs=pl.BlockSpec((blk, w), lambda i: (i, 0)),
        out_shape=jax.ShapeDtypeStruct((n, w), f32),
    )


def _mlp_kernel():
    def body(p_ref, w1_ref, b1_ref, w2_ref, b2_ref, o_ref):
        p = jnp.max(p_ref[...], axis=0)
        p = jnp.where(jnp.isfinite(p), p, 0.0)
        h = jnp.maximum(_vpu_mm(p, w1_ref, 32) + b1_ref[...], 0.0)
        o = jnp.maximum(_vpu_mm(h, w2_ref, 64) + b2_ref[...], 0.0)
        o_ref[...] = o

    return pl.pallas_call(
        body,
        out_shape=jax.ShapeDtypeStruct((G, 2), f32),
    )


def kernel(x, edge_index, batch, bn0_g, bn0_b, W1, b1, bn1_g, bn1_b,
           W2, b2, bn2_g, bn2_b, W3, b3, bn3_g, bn3_b, L1_W, L1_b, L2_W, L2_b):
    n = x.shape[0]
    e = edge_index.shape[1]

    npad = ((n + 1 + NS * 8 - 1) // (NS * 8)) * (NS * 8)
    slice_rows = npad // NS
    rows = -(-e // LANE)
    rows_pad = -(-rows // (NW * KC)) * (NW * KC)
    rpw = rows_pad // NW
    e_pad = rows_pad * LANE
    blk = 1000
    assert n % blk == 0 and n % NW == 0

    pad = e_pad - e
    src_f = jnp.concatenate([edge_index[0], jnp.zeros((pad,), i32)])
    dst_f = jnp.concatenate([edge_index[1], jnp.full((pad,), n, i32)])
    src_p = src_f.reshape(NW, rpw, LANE)
    dst_p = dst_f.reshape(NW, rpw, LANE)

    zin8 = jnp.zeros((slice_rows, 8), f32)
    zin16 = jnp.zeros((slice_rows, 16), f32)
    ones_lane = jnp.ones((LANE, 8), f32)

    deg2 = _make_deg_kernel(npad, rpw, slice_rows)(dst_p, ones_lane, zin8)
    xs = _stats_kernel(n, 1, blk)(x)
    y1, dinv = _b0_kernel(n, blk)(
        x, xs, deg2, bn0_g.reshape(1, 1), bn0_b.reshape(1, 1))

    z1 = _make_prop_kernel(npad, 8, rpw, slice_rows, "prop_l1")(
        y1, src_p, dst_p, zin8)
    g1, s1 = _a_kernel(n, 8, 1, 8, blk)(
        z1[:, :n], y1, dinv, W1, b1.reshape(1, 8))
    y2 = _b_kernel(n, 8, blk, True, True)(
        g1, s1, bn1_g.reshape(1, 8), bn1_b.reshape(1, 8), dinv)

    z2 = _make_prop_kernel(npad, 8, rpw, slice_rows, "prop_l2")(
        y2, src_p, dst_p, zin8)
    g2, s2 = _a_kernel(n, 8, 8, 16, blk)(
        z2[:, :n], y2, dinv, W2, b2.reshape(1, 16))
    y3 = _b_kernel(n, 16, blk, True, True)(
        g2, s2, bn2_g.reshape(1, 16), bn2_b.reshape(1, 16), dinv)

    z3 = _make_prop_kernel(npad, 16, rpw, slice_rows, "prop_l3")(
        y3, src_p, dst_p, zin16)
    g3, s3 = _a_kernel(n, 16, 16, 32, blk)(
        z3[:, :n], y3, dinv, W3, b3.reshape(1, 32))
    h3 = _b_kernel(n, 32, blk, True, False)(
        g3, s3, bn3_g.reshape(1, 32), bn3_b.reshape(1, 32), dinv)

    batch3 = batch.reshape(NW, 1, n // NW)
    h3r = h3.reshape(NW, n // NW, 32)
    pool_part = _make_pool_kernel(n // NW)(h3r, batch3)
    out = _mlp_kernel()(
        pool_part, L1_W, L1_b.reshape(1, 64), L2_W, L2_b.reshape(1, 2))
    return out


# speedup vs baseline: 18.7256x; 1.3279x over previous
import functools

import jax
import jax.numpy as jnp
from jax import lax
from jax.experimental import pallas as pl
from jax.experimental.pallas import tpu as pltpu
from jax.experimental.pallas import tpu_sc as plsc

EPS = 1e-5
G = 256
NC = 2
NS = 16
NW = NC * NS
LANE = 128
KC = 32

f32 = jnp.float32
i32 = jnp.int32

_SC_PARAMS = pltpu.CompilerParams(use_tc_tiling_on_sc=False)


def _mesh2():
    return plsc.VectorSubcoreMesh(core_axis_name="c", subcore_axis_name="s")


def _make_deg_kernel(npad, rpw, slice_rows):
    @functools.partial(
        pl.kernel,
        out_type=jax.ShapeDtypeStruct((NC, npad, 8), f32),
        mesh=_mesh2(),
        name="deg_hist",
        compiler_params=_SC_PARAMS,
        scratch_types=[
            pltpu.VMEM((rpw, LANE), i32),
            pltpu.VMEM((LANE, 8), f32),
            pltpu.VMEM_SHARED((npad, 8), f32),
            pltpu.SemaphoreType.DMA,
        ],
    )
    def deg_kernel(dst_hbm, ones_hbm, zin_hbm, out_hbm, dst_v, ones_v, z_sh,
                   sem_s):
        c = lax.axis_index("c")
        s = lax.axis_index("s")
        wid = c * NS + s
        pltpu.sync_copy(zin_hbm, z_sh.at[pl.ds(s * slice_rows, slice_rows), :])
        pltpu.sync_copy(ones_hbm, ones_v)
        plsc.subcore_barrier()
        pltpu.sync_copy(dst_hbm.at[wid], dst_v)

        def body(j, carry):
            sd = [pltpu.async_copy(ones_v, z_sh.at[dst_v.at[j * 8 + u]],
                                   sem_s, add=True) for u in range(8)]
            for d in sd:
                d.wait()
            return carry

        lax.fori_loop(0, rpw // 8, body, 0)
        plsc.subcore_barrier()
        pltpu.sync_copy(
            z_sh.at[pl.ds(s * slice_rows, slice_rows), :],
            out_hbm.at[c, pl.ds(s * slice_rows, slice_rows), :],
        )

    return deg_kernel


def _make_prop_kernel(npad, w, rpw, slice_rows, name, kcp):
    nchunks = rpw // kcp

    @functools.partial(
        pl.kernel,
        out_type=jax.ShapeDtypeStruct((NC, npad, w), f32),
        mesh=_mesh2(),
        name=name,
        compiler_params=_SC_PARAMS,
        scratch_types=[
            pltpu.VMEM((kcp, LANE), i32),
            pltpu.VMEM((kcp, LANE), i32),
            pltpu.VMEM((kcp, LANE, w), f32),
            pltpu.VMEM_SHARED((npad, w), f32),
            pltpu.SemaphoreType.DMA,
            pltpu.SemaphoreType.DMA,
        ],
    )
    def prop_kernel(y_hbm, src_hbm, dst_hbm, zin_hbm, out_hbm,
                    src_v, dst_v, rows_v, z_sh, sem_g, sem_s):
        c = lax.axis_index("c")
        s = lax.axis_index("s")
        wid = c * NS + s
        pltpu.sync_copy(zin_hbm, z_sh.at[pl.ds(s * slice_rows, slice_rows), :])
        plsc.subcore_barrier()

        def chunk(k, carry):
            pltpu.sync_copy(src_hbm.at[wid, pl.ds(k * kcp, kcp)], src_v)
            pltpu.sync_copy(dst_hbm.at[wid, pl.ds(k * kcp, kcp)], dst_v)
            gd = [pltpu.async_copy(y_hbm.at[src_v.at[j]], rows_v.at[j], sem_g)
                  for j in range(kcp)]
            sd = []
            for j in range(kcp):
                gd[j].wait()
                sd.append(pltpu.async_copy(
                    rows_v.at[j], z_sh.at[dst_v.at[j]], sem_s, add=True))
            for d in sd:
                d.wait()
            return carry

        lax.fori_loop(0, nchunks, chunk, 0)
        plsc.subcore_barrier()
        pltpu.sync_copy(
            z_sh.at[pl.ds(s * slice_rows, slice_rows), :],
            out_hbm.at[c, pl.ds(s * slice_rows, slice_rows), :],
        )

    return prop_kernel


def _make_pool_kernel(rows_per_worker):
    rw = rows_per_worker

    @functools.partial(
        pl.kernel,
        out_type=jax.ShapeDtypeStruct((NW, G, 32), f32),
        mesh=_mesh2(),
        name="pool_max",
        compiler_params=_SC_PARAMS,
        scratch_types=[
            pltpu.VMEM((rw + 16,), i32),
            pltpu.VMEM((rw, 32), f32),
            pltpu.VMEM((G, 32), f32),
        ],
    )
    def pool_kernel(h_hbm, batch_hbm, out_hbm, ids_v, h_v, acc_v):
        c = lax.axis_index("c")
        s = lax.axis_index("s")
        wid = c * NS + s
        pltpu.sync_copy(batch_hbm.at[wid, 0], ids_v.at[pl.ds(0, rw)])
        pltpu.sync_copy(h_hbm.at[wid], h_v)

        neg = jnp.full((16,), -jnp.inf, f32)

        def init(r, carry):
            acc_v[r, pl.ds(0, 16)] = neg
            acc_v[r, pl.ds(16, 16)] = neg
            return carry

        lax.fori_loop(0, G, init, 0)

        def body(r, carry):
            g = ids_v[pl.ds(r, 16)][0]
            a0 = acc_v[g, pl.ds(0, 16)]
            a1 = acc_v[g, pl.ds(16, 16)]
            h0 = h_v[r, pl.ds(0, 16)]
            h1 = h_v[r, pl.ds(16, 16)]
            acc_v[g, pl.ds(0, 16)] = jnp.maximum(a0, h0)
            acc_v[g, pl.ds(16, 16)] = jnp.maximum(a1, h1)
            return carry

        lax.fori_loop(0, rw, body, 0)
        pltpu.sync_copy(acc_v, out_hbm.at[wid])

    return pool_kernel


def _vpu_mm(a, w_ref, k):
    acc = a[:, 0:1] * w_ref[0, :][None, :]
    for j in range(1, k):
        acc = acc + a[:, j:j + 1] * w_ref[j, :][None, :]
    return acc


def _stats_kernel(n, w, blk):
    nb = n // blk

    def body(x_ref, s_ref):
        i = pl.program_id(0)

        @pl.when(i == 0)
        def _():
            s_ref[...] = jnp.zeros_like(s_ref)

        xb = x_ref[...]
        ssum = jnp.sum(xb, axis=0, keepdims=True)
        ssq = jnp.sum(xb * xb, axis=0, keepdims=True)
        s_ref[...] += jnp.concatenate([ssum, ssq], axis=0)

    return pl.pallas_call(
        body,
        grid=(nb,),
        in_specs=[pl.BlockSpec((blk, w), lambda i: (i, 0))],
        out_specs=pl.BlockSpec((2, w), lambda i: (0, 0)),
        out_shape=jax.ShapeDtypeStruct((2, w), f32),
    )


def _b0_kernel(n, blk):
    nb = n // blk

    def body(x_ref, xs_ref, deg_ref, g_ref, b_ref, y_ref, dinv_ref):
        nn = f32(n)
        m = xs_ref[0, 0] / nn
        v = xs_ref[1, 0] / nn - m * m
        d = deg_ref[0][:, 0:1] + deg_ref[1][:, 0:1] + 1.0
        dinv = lax.rsqrt(d)
        h = g_ref[0, 0] * (x_ref[...] - m) * lax.rsqrt(v + EPS) + b_ref[0, 0]
        dinv_ref[...] = dinv
        y = dinv * h
        y_ref[...] = jnp.concatenate(
            [y, jnp.zeros((y.shape[0], 7), f32)], axis=1)

    return pl.pallas_call(
        body,
        grid=(nb,),
        in_specs=[
            pl.BlockSpec((blk, 1), lambda i: (i, 0)),
            pl.BlockSpec((2, 1), lambda i: (0, 0)),
            pl.BlockSpec((2, blk, 8), lambda i: (0, i, 0)),
            pl.BlockSpec((1, 1), lambda i: (0, 0)),
            pl.BlockSpec((1, 1), lambda i: (0, 0)),
        ],
        out_specs=[
            pl.BlockSpec((blk, 8), lambda i: (i, 0)),
            pl.BlockSpec((blk, 1), lambda i: (i, 0)),
        ],
        out_shape=[
            jax.ShapeDtypeStruct((n, 8), f32),
            jax.ShapeDtypeStruct((n, 1), f32),
        ],
    )


def _a_kernel(n, w_in, w_use, w_out, blk):
    nb = n // blk

    def body(z_ref, y_ref, dinv_ref, w_ref, b_ref, g_ref, s_ref):
        i = pl.program_id(0)
        a = dinv_ref[...] * (z_ref[0] + z_ref[1] + y_ref[...])
        g = _vpu_mm(a, w_ref, w_use) + b_ref[...]
        g_ref[...] = g

        @pl.when(i == 0)
        def _():
            s_ref[...] = jnp.zeros_like(s_ref)

        ssum = jnp.sum(g, axis=0, keepdims=True)
        ssq = jnp.sum(g * g, axis=0, keepdims=True)
        s_ref[...] += jnp.concatenate([ssum, ssq], axis=0)

    return pl.pallas_call(
        body,
        grid=(nb,),
        in_specs=[
            pl.BlockSpec((2, blk, w_in), lambda i: (0, i, 0)),
            pl.BlockSpec((blk, w_in), lambda i: (i, 0)),
            pl.BlockSpec((blk, 1), lambda i: (i, 0)),
            pl.BlockSpec((w_use, w_out), lambda i: (0, 0)),
            pl.BlockSpec((1, w_out), lambda i: (0, 0)),
        ],
        out_specs=[
            pl.BlockSpec((blk, w_out), lambda i: (i, 0)),
            pl.BlockSpec((2, w_out), lambda i: (0, 0)),
        ],
        out_shape=[
            jax.ShapeDtypeStruct((n, w_out), f32),
            jax.ShapeDtypeStruct((2, w_out), f32),
        ],
    )


def _b_kernel(n, w, blk, relu, dinv_scale):
    nb = n // blk

    def body(g_ref, s_ref, gam_ref, bet_ref, dinv_ref, y_ref):
        nn = f32(n)
        m = s_ref[0:1, :] / nn
        v = s_ref[1:2, :] / nn - m * m
        h = gam_ref[...] * (g_ref[...] - m) * lax.rsqrt(v + EPS) + bet_ref[...]
        if relu:
            h = jnp.maximum(h, 0.0)
        if dinv_scale:
            h = dinv_ref[...] * h
        y_ref[...] = h

    return pl.pallas_call(
        body,
        grid=(nb,),
        in_specs=[
            pl.BlockSpec((blk, w), lambda i: (i, 0)),
            pl.BlockSpec((2, w), lambda i: (0, 0)),
            pl.BlockSpec((1, w), lambda i: (0, 0)),
            pl.BlockSpec((1, w), lambda i: (0, 0)),
            pl.BlockSpec((blk, 1), lambda i: (i, 0)),
        ],
        out_specs=pl.BlockSpec((blk, w), lambda i: (i, 0)),
        out_shape=jax.ShapeDtypeStruct((n, w), f32),
    )


def _mlp_kernel():
    def body(p_ref, w1_ref, b1_ref, w2_ref, b2_ref, o_ref):
        p = jnp.max(p_ref[...], axis=0)
        p = jnp.where(jnp.isfinite(p), p, 0.0)
        h = jnp.maximum(_vpu_mm(p, w1_ref, 32) + b1_ref[...], 0.0)
        o = jnp.maximum(_vpu_mm(h, w2_ref, 64) + b2_ref[...], 0.0)
        o_ref[...] = o

    return pl.pallas_call(
        body,
        out_shape=jax.ShapeDtypeStruct((G, 2), f32),
    )


def kernel(x, edge_index, batch, bn0_g, bn0_b, W1, b1, bn1_g, bn1_b,
           W2, b2, bn2_g, bn2_b, W3, b3, bn3_g, bn3_b, L1_W, L1_b, L2_W, L2_b):
    n = x.shape[0]
    e = edge_index.shape[1]

    npad = ((n + 1 + NS * 8 - 1) // (NS * 8)) * (NS * 8)
    slice_rows = npad // NS
    rows = -(-e // LANE)
    rows_pad = -(-rows // (NW * 16)) * (NW * 16)
    rpw = rows_pad // NW
    e_pad = rows_pad * LANE
    blk = 1000
    assert n % blk == 0 and n % NW == 0

    pad = e_pad - e
    src_f = jnp.concatenate([edge_index[0], jnp.zeros((pad,), i32)])
    dst_f = jnp.concatenate([edge_index[1], jnp.full((pad,), n, i32)])
    src_p = src_f.reshape(NW, rpw, LANE)
    dst_p = dst_f.reshape(NW, rpw, LANE)

    zin8 = jnp.zeros((slice_rows, 8), f32)
    zin16 = jnp.zeros((slice_rows, 16), f32)
    ones_lane = jnp.ones((LANE, 8), f32)

    deg2 = _make_deg_kernel(npad, rpw, slice_rows)(dst_p, ones_lane, zin8)
    xs = _stats_kernel(n, 1, blk)(x)
    y1, dinv = _b0_kernel(n, blk)(
        x, xs, deg2, bn0_g.reshape(1, 1), bn0_b.reshape(1, 1))

    z1 = _make_prop_kernel(npad, 8, rpw, slice_rows, "prop_l1", 16)(
        y1, src_p, dst_p, zin8)
    g1, s1 = _a_kernel(n, 8, 1, 8, blk)(
        z1[:, :n], y1, dinv, W1, b1.reshape(1, 8))
    y2 = _b_kernel(n, 8, blk, True, True)(
        g1, s1, bn1_g.reshape(1, 8), bn1_b.reshape(1, 8), dinv)

    z2 = _make_prop_kernel(npad, 8, rpw, slice_rows, "prop_l2", 16)(
        y2, src_p, dst_p, zin8)
    g2, s2 = _a_kernel(n, 8, 8, 16, blk)(
        z2[:, :n], y2, dinv, W2, b2.reshape(1, 16))
    y3 = _b_kernel(n, 16, blk, True, True)(
        g2, s2, bn2_g.reshape(1, 16), bn2_b.reshape(1, 16), dinv)

    z3 = _make_prop_kernel(npad, 16, rpw, slice_rows, "prop_l3", 8)(
        y3, src_p, dst_p, zin16)
    g3, s3 = _a_kernel(n, 16, 16, 32, blk)(
        z3[:, :n], y3, dinv, W3, b3.reshape(1, 32))
    h3 = _b_kernel(n, 32, blk, True, False)(
        g3, s3, bn3_g.reshape(1, 32), bn3_b.reshape(1, 32), dinv)

    batch3 = batch.reshape(NW, 1, n // NW)
    h3r = h3.reshape(NW, n // NW, 32)
    pool_part = _make_pool_kernel(n // NW)(h3r, batch3)
    out = _mlp_kernel()(
        pool_part, L1_W, L1_b.reshape(1, 64), L2_W, L2_b.reshape(1, 2))
    return out
